# Initial kernel scaffold; baseline (speedup 1.0000x reference)
import jax
import jax.numpy as jnp
from jax.experimental import pallas as pl


def kernel(indices, W):
    raise NotImplementedError("write your pallas kernel here")


# speedup vs baseline: 3.9616x; 3.9616x over previous
import functools

import jax
import jax.numpy as jnp
from jax import lax
from jax.experimental import pallas as pl
from jax.experimental.pallas import tpu as pltpu
from jax.experimental.pallas import tpu_sc as plsc

B = 16384
L = 200
VOCAB = 10
DIM = 5
N = B * L
LANES = 16
NW = 32
ROWS_PER_W = N // NW
CH = 4096
NCHUNK = ROWS_PER_W // CH
BLKS = CH // LANES


def _full(v):
    return jnp.full((LANES,), v, jnp.int32)


def _sc_body(idx_hbm, wt_hbm, out_hbm, wt_v, g_v, idx_v, out_v):
    wid = lax.axis_index("s") * 2 + lax.axis_index("c")
    base = wid * ROWS_PER_W

    pltpu.sync_copy(wt_hbm, wt_v.at[pl.ds(LANES, LANES * LANES)])
    for i in range(VOCAB):
        acc = None
        for d in range(DIM):
            w_id = plsc.load_gather(wt_v, [_full(LANES + LANES * d + i)])
            term = w_id * wt_v[pl.ds(LANES + LANES * d, LANES)]
            acc = term if acc is None else acc + term
        g_v[pl.ds(LANES * i, LANES)] = acc

    iota = lax.iota(jnp.int32, LANES)
    pats, cols = [], []
    for j in range(VOCAB):
        q = iota + (LANES * j)
        r = q // VOCAB
        pats.append(r)
        cols.append(q - r * VOCAB)

    def chunk_body(c, carry):
        start = base + c * CH
        pltpu.sync_copy(idx_hbm.at[pl.ds(start, CH)], idx_v)

        def blk_body(b, carry2):
            rbase = b * LANES
            obase = b * (LANES * VOCAB)
            for j in range(VOCAB):
                rows = plsc.load_gather(idx_v, [pats[j] + rbase])
                flat = lax.shift_left(rows, 4) + cols[j]
                vals = plsc.load_gather(g_v, [flat])
                out_v[pl.ds(obase + LANES * j, LANES)] = vals
            return carry2

        lax.fori_loop(0, BLKS, blk_body, 0)
        pltpu.sync_copy(out_v, out_hbm.at[pl.ds(start * VOCAB, CH * VOCAB)])
        return carry

    lax.fori_loop(0, NCHUNK, chunk_body, 0)


_emb_gather = functools.partial(
    pl.kernel,
    mesh=plsc.VectorSubcoreMesh(core_axis_name="c", subcore_axis_name="s"),
    out_type=jax.ShapeDtypeStruct((N * VOCAB,), jnp.float32),
    compiler_params=pltpu.CompilerParams(needs_layout_passes=False),
    scratch_types=[
        pltpu.VMEM((LANES * LANES + LANES,), jnp.float32),
        pltpu.VMEM((LANES * LANES,), jnp.float32),
        pltpu.VMEM((CH,), jnp.int32),
        pltpu.VMEM((CH * VOCAB,), jnp.float32),
    ],
)(_sc_body)


def kernel(indices, W):
    idx = indices.reshape(-1).astype(jnp.int32)
    wt = jnp.zeros((LANES, LANES), jnp.float32).at[:DIM, :VOCAB].set(W.T).reshape(-1)
    out = _emb_gather(idx, wt)
    return out.reshape(B, L, VOCAB)


# speedup vs baseline: 4.8055x; 1.2130x over previous
import functools

import jax
import jax.numpy as jnp
from jax import lax
from jax.experimental import pallas as pl
from jax.experimental.pallas import tpu as pltpu
from jax.experimental.pallas import tpu_sc as plsc

B = 16384
L = 200
VOCAB = 10
DIM = 5
N = B * L
LANES = 16
NW = 32
ROWS_PER_W = N // NW
CH = 5120
NCHUNK = ROWS_PER_W // CH
BLKS = CH // LANES


def _full(v):
    return jnp.full((LANES,), v, jnp.int32)


def _sc_body(idx_hbm, wt_hbm, out_hbm,
             wt_v, g_v, idx_v0, idx_v1, out_v0, out_v1,
             sem_i0, sem_i1, sem_o0, sem_o1):
    wid = lax.axis_index("s") * 2 + lax.axis_index("c")
    base = wid * ROWS_PER_W
    idx_bufs = (idx_v0, idx_v1)
    out_bufs = (out_v0, out_v1)
    sem_i = (sem_i0, sem_i1)
    sem_o = (sem_o0, sem_o1)

    pltpu.sync_copy(wt_hbm, wt_v.at[pl.ds(LANES, LANES * LANES)])
    for i in range(VOCAB):
        acc = None
        for d in range(DIM):
            w_id = plsc.load_gather(wt_v, [_full(LANES + LANES * d + i)])
            term = w_id * wt_v[pl.ds(LANES + LANES * d, LANES)]
            acc = term if acc is None else acc + term
        g_v[pl.ds(LANES * i, LANES)] = acc

    iota = lax.iota(jnp.int32, LANES)
    pats, cols = [], []
    for j in range(VOCAB):
        q = iota + (LANES * j)
        r = q // VOCAB
        pats.append(r)
        cols.append(q - r * VOCAB)

    def compute_chunk(idx_v, out_v):
        @plsc.parallel_loop(0, BLKS, unroll=8)
        def _(b):
            rbase = b * LANES
            obase = b * (LANES * VOCAB)
            for j in range(VOCAB):
                rows = plsc.load_gather(idx_v, [pats[j] + rbase])
                flat = lax.shift_left(rows, 4) + cols[j]
                vals = plsc.load_gather(g_v, [flat])
                out_v[pl.ds(obase + LANES * j, LANES)] = vals

    def wait_idx(b):
        pltpu.make_async_copy(
            idx_hbm.at[pl.ds(base, CH)], idx_bufs[b], sem_i[b]).wait()

    def wait_out(b):
        pltpu.make_async_copy(
            out_bufs[b], out_hbm.at[pl.ds(base * VOCAB, CH * VOCAB)],
            sem_o[b]).wait()

    pltpu.async_copy(idx_hbm.at[pl.ds(base, CH)], idx_bufs[0], sem_i[0])
    pltpu.async_copy(idx_hbm.at[pl.ds(base + CH, CH)], idx_bufs[1], sem_i[1])

    def pair_body(g, carry):
        for b in range(2):
            c = 2 * g + b
            wait_idx(b)

            @pl.when(c >= 2)
            def _():
                wait_out(b)

            compute_chunk(idx_bufs[b], out_bufs[b])
            start = base + c * CH
            pltpu.async_copy(
                out_bufs[b], out_hbm.at[pl.ds(start * VOCAB, CH * VOCAB)],
                sem_o[b])

            @pl.when(c + 2 < NCHUNK)
            def _():
                start_n = base + (c + 2) * CH
                pltpu.async_copy(
                    idx_hbm.at[pl.ds(start_n, CH)], idx_bufs[b], sem_i[b])
        return carry

    lax.fori_loop(0, NCHUNK // 2, pair_body, 0)
    wait_out(0)
    wait_out(1)


_emb_gather = functools.partial(
    pl.kernel,
    mesh=plsc.VectorSubcoreMesh(core_axis_name="c", subcore_axis_name="s"),
    out_type=jax.ShapeDtypeStruct((N * VOCAB,), jnp.float32),
    compiler_params=pltpu.CompilerParams(needs_layout_passes=False),
    scratch_types=[
        pltpu.VMEM((LANES * LANES + LANES,), jnp.float32),
        pltpu.VMEM((LANES * LANES,), jnp.float32),
        pltpu.VMEM((CH,), jnp.int32),
        pltpu.VMEM((CH,), jnp.int32),
        pltpu.VMEM((CH * VOCAB,), jnp.float32),
        pltpu.VMEM((CH * VOCAB,), jnp.float32),
        pltpu.SemaphoreType.DMA,
        pltpu.SemaphoreType.DMA,
        pltpu.SemaphoreType.DMA,
        pltpu.SemaphoreType.DMA,
    ],
)(_sc_body)


def kernel(indices, W):
    idx = indices.reshape(-1).astype(jnp.int32)
    wt = jnp.zeros((LANES, LANES), jnp.float32).at[:DIM, :VOCAB].set(W.T).reshape(-1)
    out = _emb_gather(idx, wt)
    return out.reshape(B, L, VOCAB)


# speedup vs baseline: 8.9976x; 1.8724x over previous
import functools

import jax
import jax.numpy as jnp
from jax import lax
from jax.experimental import pallas as pl
from jax.experimental.pallas import tpu as pltpu
from jax.experimental.pallas import tpu_sc as plsc

B = 16384
L = 200
VOCAB = 10
DIM = 5
N = B * L
LANES = 16
NW = 32
B_PER_W = B // NW
KB = 2
CH = KB * L
NCHUNK = B_PER_W // KB
ROWS_PER_W = B_PER_W * L
BLKS = CH // LANES


def _full(v):
    return jnp.full((LANES,), v, jnp.int32)


def _sc_body(idx_hbm, wt_hbm, out_hbm,
             wt_v, g_v, idx_v0, idx_v1, out_v0, out_v1,
             sem_i0, sem_i1, sem_o0, sem_o1):
    wid = lax.axis_index("s") * 2 + lax.axis_index("c")
    base = wid * ROWS_PER_W
    b0w = wid * B_PER_W
    idx_bufs = (idx_v0, idx_v1)
    out_bufs = (out_v0, out_v1)
    sem_i = (sem_i0, sem_i1)
    sem_o = (sem_o0, sem_o1)

    pltpu.sync_copy(wt_hbm, wt_v.at[pl.ds(LANES, LANES * LANES)])
    for i in range(VOCAB):
        acc = None
        for d in range(DIM):
            w_id = plsc.load_gather(wt_v, [_full(LANES + LANES * d + i)])
            term = w_id * wt_v[pl.ds(LANES + LANES * d, LANES)]
            acc = term if acc is None else acc + term
        g_v[pl.ds(LANES * i, LANES)] = acc

    iota = lax.iota(jnp.int32, LANES)
    pats, cols = [], []
    for j in range(VOCAB):
        q = iota + (LANES * j)
        r = q // VOCAB
        pats.append(r)
        cols.append(q - r * VOCAB)

    def compute_chunk(idx_v, out_v):
        @plsc.parallel_loop(0, BLKS, unroll=8)
        def _(b):
            rbase = b * LANES
            for j in range(VOCAB):
                rowsel = pats[j] + rbase
                rows = plsc.load_gather(idx_v, [rowsel])
                flat = lax.shift_left(rows, 4) + cols[j]
                vals = plsc.load_gather(g_v, [flat])
                plsc.store_scatter(out_v, [rowsel, cols[j]], vals)

    def wait_idx(b):
        pltpu.make_async_copy(
            idx_hbm.at[pl.ds(base, CH)], idx_bufs[b], sem_i[b]).wait()

    def out_dst(c):
        return out_hbm.at[pl.ds(b0w + c * KB, KB)]

    def wait_out(b):
        pltpu.make_async_copy(
            out_bufs[b].reshape(KB, L, VOCAB), out_dst(0), sem_o[b]).wait()

    pltpu.async_copy(idx_hbm.at[pl.ds(base, CH)], idx_bufs[0], sem_i[0])
    pltpu.async_copy(idx_hbm.at[pl.ds(base + CH, CH)], idx_bufs[1], sem_i[1])

    def pair_body(g, carry):
        for b in range(2):
            c = 2 * g + b
            wait_idx(b)

            @pl.when(c >= 2)
            def _():
                wait_out(b)

            compute_chunk(idx_bufs[b], out_bufs[b])
            pltpu.async_copy(
                out_bufs[b].reshape(KB, L, VOCAB), out_dst(c), sem_o[b])

            @pl.when(c + 2 < NCHUNK)
            def _():
                start_n = base + (c + 2) * CH
                pltpu.async_copy(
                    idx_hbm.at[pl.ds(start_n, CH)], idx_bufs[b], sem_i[b])
        return carry

    lax.fori_loop(0, NCHUNK // 2, pair_body, 0)
    wait_out(0)
    wait_out(1)


_emb_gather = functools.partial(
    pl.kernel,
    mesh=plsc.VectorSubcoreMesh(core_axis_name="c", subcore_axis_name="s"),
    out_type=jax.ShapeDtypeStruct((B, L, VOCAB), jnp.float32),
    compiler_params=pltpu.CompilerParams(needs_layout_passes=False),
    scratch_types=[
        pltpu.VMEM((LANES * LANES + LANES,), jnp.float32),
        pltpu.VMEM((LANES * LANES,), jnp.float32),
        pltpu.VMEM((CH,), jnp.int32),
        pltpu.VMEM((CH,), jnp.int32),
        pltpu.VMEM((CH, VOCAB), jnp.float32),
        pltpu.VMEM((CH, VOCAB), jnp.float32),
        pltpu.SemaphoreType.DMA,
        pltpu.SemaphoreType.DMA,
        pltpu.SemaphoreType.DMA,
        pltpu.SemaphoreType.DMA,
    ],
)(_sc_body)


def kernel(indices, W):
    idx = indices.reshape(-1).astype(jnp.int32)
    wt = jnp.zeros((LANES, LANES), jnp.float32).at[:DIM, :VOCAB].set(W.T).reshape(-1)
    return _emb_gather(idx, wt)


# speedup vs baseline: 9.0775x; 1.0089x over previous
import functools

import jax
import jax.numpy as jnp
from jax import lax
from jax.experimental import pallas as pl
from jax.experimental.pallas import tpu as pltpu
from jax.experimental.pallas import tpu_sc as plsc

B = 16384
L = 200
VOCAB = 10
DIM = 5
N = B * L
LANES = 16
NW = 32
B_PER_W = B // NW
KB = 2
CH = KB * L
NCHUNK = B_PER_W // KB
ROWS_PER_W = B_PER_W * L
BLKS = CH // LANES


def _full(v):
    return jnp.full((LANES,), v, jnp.int32)


def _sc_body(idx_hbm, wt_hbm, out_hbm,
             wt_v, g_v, idx_v0, idx_v1, out_v0, out_v1,
             sem_i0, sem_i1, sem_o0, sem_o1):
    wid = lax.axis_index("s") * 2 + lax.axis_index("c")
    base = wid * ROWS_PER_W
    b0w = wid * B_PER_W
    idx_bufs = (idx_v0, idx_v1)
    out_bufs = (out_v0, out_v1)
    sem_i = (sem_i0, sem_i1)
    sem_o = (sem_o0, sem_o1)

    pltpu.sync_copy(wt_hbm, wt_v.at[pl.ds(LANES, LANES * LANES)])
    for i in range(VOCAB):
        acc = None
        for d in range(DIM):
            w_id = plsc.load_gather(wt_v, [_full(LANES + LANES * d + i)])
            term = w_id * wt_v[pl.ds(LANES + LANES * d, LANES)]
            acc = term if acc is None else acc + term
        g_v[pl.ds(LANES * i, LANES)] = acc

    iota = lax.iota(jnp.int32, LANES)
    pats, cols = [], []
    for j in range(VOCAB):
        q = iota + (LANES * j)
        r = q // VOCAB
        pats.append(r)
        cols.append(q - r * VOCAB)

    def compute_chunk(idx_v, out_v):
        @plsc.parallel_loop(0, BLKS, unroll=8)
        def _(b):
            rbase = b * LANES
            for j in range(VOCAB):
                rowsel = pats[j] + rbase
                rmaj = rowsel // L
                rmin = rowsel - rmaj * L
                rows = plsc.load_gather(idx_v, [rmaj, rmin])
                flat = lax.shift_left(rows, 4) + cols[j]
                vals = plsc.load_gather(g_v, [flat])
                plsc.store_scatter(out_v, [rowsel, cols[j]], vals)

    def idx_dst(b):
        return idx_bufs[b]

    def wait_idx(b):
        pltpu.make_async_copy(
            idx_hbm.at[pl.ds(b0w, KB)], idx_dst(b), sem_i[b]).wait()

    def out_dst(c):
        return out_hbm.at[pl.ds(b0w + c * KB, KB)]

    def wait_out(b):
        pltpu.make_async_copy(
            out_bufs[b].reshape(KB, L, VOCAB), out_dst(0), sem_o[b]).wait()

    pltpu.async_copy(idx_hbm.at[pl.ds(b0w, KB)], idx_dst(0), sem_i[0])
    pltpu.async_copy(idx_hbm.at[pl.ds(b0w + KB, KB)], idx_dst(1), sem_i[1])

    def pair_body(g, carry):
        for b in range(2):
            c = 2 * g + b
            wait_idx(b)

            @pl.when(c >= 2)
            def _():
                wait_out(b)

            compute_chunk(idx_bufs[b], out_bufs[b])
            pltpu.async_copy(
                out_bufs[b].reshape(KB, L, VOCAB), out_dst(c), sem_o[b])

            @pl.when(c + 2 < NCHUNK)
            def _():
                bn = b0w + (c + 2) * KB
                pltpu.async_copy(
                    idx_hbm.at[pl.ds(bn, KB)], idx_dst(b), sem_i[b])
        return carry

    lax.fori_loop(0, NCHUNK // 2, pair_body, 0)
    wait_out(0)
    wait_out(1)


_emb_gather = functools.partial(
    pl.kernel,
    mesh=plsc.VectorSubcoreMesh(core_axis_name="c", subcore_axis_name="s"),
    out_type=jax.ShapeDtypeStruct((B, L, VOCAB), jnp.float32),
    compiler_params=pltpu.CompilerParams(needs_layout_passes=False),
    scratch_types=[
        pltpu.VMEM((LANES * LANES + LANES,), jnp.float32),
        pltpu.VMEM((LANES * LANES,), jnp.float32),
        pltpu.VMEM((KB, L), jnp.int32),
        pltpu.VMEM((KB, L), jnp.int32),
        pltpu.VMEM((CH, VOCAB), jnp.float32),
        pltpu.VMEM((CH, VOCAB), jnp.float32),
        pltpu.SemaphoreType.DMA,
        pltpu.SemaphoreType.DMA,
        pltpu.SemaphoreType.DMA,
        pltpu.SemaphoreType.DMA,
    ],
)(_sc_body)


def kernel(indices, W):
    idx = indices.astype(jnp.int32)
    wt = jnp.zeros((LANES, LANES), jnp.float32).at[:DIM, :VOCAB].set(W.T).reshape(-1)
    return _emb_gather(idx, wt)


# speedup vs baseline: 9.1063x; 1.0032x over previous
import functools

import jax
import jax.numpy as jnp
from jax import lax
from jax.experimental import pallas as pl
from jax.experimental.pallas import tpu as pltpu
from jax.experimental.pallas import tpu_sc as plsc

B = 16384
L = 200
VOCAB = 10
DIM = 5
N = B * L
LANES = 16
NW = 32
B_PER_W = B // NW
KB = 2
CH = KB * L
NCHUNK = B_PER_W // KB
ROWS_PER_W = B_PER_W * L
BLKS = CH // LANES


def _full(v):
    return jnp.full((LANES,), v, jnp.int32)


def _sc_body(idx_hbm, wt_hbm, out_hbm,
             wt_v, g_v, idx_v0, idx_v1, out_v0, out_v1,
             sem_i0, sem_i1, sem_o0, sem_o1):
    wid = lax.axis_index("s") * 2 + lax.axis_index("c")
    b0w = wid * B_PER_W
    idx_bufs = (idx_v0, idx_v1)
    out_bufs = (out_v0, out_v1)
    sem_i = (sem_i0, sem_i1)
    sem_o = (sem_o0, sem_o1)

    pltpu.sync_copy(wt_hbm, wt_v.at[pl.ds(LANES, LANES * LANES)])
    for i in range(VOCAB):
        acc = None
        for d in range(DIM):
            w_id = plsc.load_gather(wt_v, [_full(LANES + LANES * d + i)])
            term = w_id * wt_v[pl.ds(LANES + LANES * d, LANES)]
            acc = term if acc is None else acc + term
        g_v[pl.ds(LANES * i, LANES)] = acc

    iota = lax.iota(jnp.int32, LANES)
    pats, cols = [], []
    for j in range(VOCAB):
        q = iota + (LANES * j)
        r = q // VOCAB
        pats.append(r)
        cols.append(q - r * VOCAB)

    def compute_chunk(idx_v, out_v):
        @plsc.parallel_loop(0, BLKS, unroll=5)
        def _(b):
            rbase = b * LANES
            for j in range(VOCAB):
                rowsel = pats[j] + rbase
                rmaj = rowsel // L
                rmin = rowsel - rmaj * L
                rows = plsc.load_gather(idx_v, [rmaj, rmin])
                flat = lax.shift_left(rows, 4) + cols[j]
                vals = plsc.load_gather(g_v, [flat])
                plsc.store_scatter(out_v, [rowsel, cols[j]], vals)

    def idx_dst(b):
        return idx_bufs[b]

    def wait_idx(b):
        pltpu.make_async_copy(
            idx_hbm.at[pl.ds(b0w, KB)], idx_dst(b), sem_i[b]).wait()

    def out_dst(c):
        return out_hbm.at[pl.ds(b0w + c * KB, KB)]

    def wait_out(b):
        pltpu.make_async_copy(
            out_bufs[b].reshape(KB, L, VOCAB), out_dst(0), sem_o[b]).wait()

    pltpu.async_copy(idx_hbm.at[pl.ds(b0w, KB)], idx_dst(0), sem_i[0])
    pltpu.async_copy(idx_hbm.at[pl.ds(b0w + KB, KB)], idx_dst(1), sem_i[1])

    def pair_body(g, carry):
        for b in range(2):
            c = 2 * g + b
            wait_idx(b)

            @pl.when(c >= 2)
            def _():
                wait_out(b)

            compute_chunk(idx_bufs[b], out_bufs[b])
            pltpu.async_copy(
                out_bufs[b].reshape(KB, L, VOCAB), out_dst(c), sem_o[b])

            @pl.when(c + 2 < NCHUNK)
            def _():
                bn = b0w + (c + 2) * KB
                pltpu.async_copy(
                    idx_hbm.at[pl.ds(bn, KB)], idx_dst(b), sem_i[b])
        return carry

    lax.fori_loop(0, NCHUNK // 2, pair_body, 0)
    wait_out(0)
    wait_out(1)


_emb_gather = functools.partial(
    pl.kernel,
    mesh=plsc.VectorSubcoreMesh(core_axis_name="c", subcore_axis_name="s"),
    out_type=jax.ShapeDtypeStruct((B, L, VOCAB), jnp.float32),
    compiler_params=pltpu.CompilerParams(needs_layout_passes=False),
    scratch_types=[
        pltpu.VMEM((LANES * LANES + LANES,), jnp.float32),
        pltpu.VMEM((LANES * LANES,), jnp.float32),
        pltpu.VMEM((KB, L), jnp.int32),
        pltpu.VMEM((KB, L), jnp.int32),
        pltpu.VMEM((CH, VOCAB), jnp.float32),
        pltpu.VMEM((CH, VOCAB), jnp.float32),
        pltpu.SemaphoreType.DMA,
        pltpu.SemaphoreType.DMA,
        pltpu.SemaphoreType.DMA,
        pltpu.SemaphoreType.DMA,
    ],
)(_sc_body)


def kernel(indices, W):
    idx = indices.astype(jnp.int32)
    wt = jnp.zeros((LANES, LANES), jnp.float32).at[:DIM, :VOCAB].set(W.T).reshape(-1)
    return _emb_gather(idx, wt)
